# baseline (speedup 1.0000x reference)
import functools

import jax
import jax.numpy as jnp
from jax import lax
from jax.experimental import pallas as pl
from jax.experimental.pallas import tpu as pltpu
from jax.experimental.pallas import tpu_sc as plsc

B = 16384
S = 200
D = 64
NC = 2
NS = 16
NW = NC * NS
BPW = B // NW
CB = 4
NCH = BPW // CB
G = 80
NG = (CB * S) // G
L = 16
NVR = D // L

_mesh = plsc.VectorSubcoreMesh(core_axis_name="c", subcore_axis_name="s")

V = 1000000
TBLK = 128
NBLK = V // TBLK
VTAIL = NBLK * TBLK
BLK_PER_W = NBLK // NW
BLK_REM = NBLK % NW


@functools.partial(
    pl.kernel,
    mesh=_mesh,
    out_type=jax.ShapeDtypeStruct((V * D,), jnp.float32),
    scratch_types=[
        pltpu.VMEM((2, D, TBLK), jnp.float32),
        pltpu.VMEM((2, TBLK * D), jnp.float32),
        pltpu.VMEM(((V - VTAIL) * D,), jnp.float32),
        pltpu.SemaphoreType.DMA((2,)),
        pltpu.SemaphoreType.DMA((2,)),
    ],
    compiler_params=pltpu.CompilerParams(
        use_tc_tiling_on_sc=True, needs_layout_passes=False
    ),
)
def _table_lin(tt_hbm, tail_hbm, out_hbm, slab_v, tslab_v, tail_v, in_sem, out_sem):
    wid = lax.axis_index("s") * NC + lax.axis_index("c")
    base = wid * BLK_PER_W + jnp.minimum(wid, BLK_REM)
    cnt = BLK_PER_W + jnp.where(wid < BLK_REM, 1, 0)
    fidx = [
        lax.broadcasted_iota(jnp.int32, (L,), 0) + L * l for l in range(NVR)
    ]

    def in_copy(c, sl):
        return pltpu.make_async_copy(
            tt_hbm.at[:, pl.ds(c * TBLK, TBLK)], slab_v.at[sl], in_sem.at[sl]
        )

    def out_copy(c, sl):
        return pltpu.make_async_copy(
            tslab_v.at[sl],
            out_hbm.at[pl.ds(c * (TBLK * D), TBLK * D)],
            out_sem.at[sl],
        )

    in_copy(base, 0).start()

    def blk(t, carry):
        sl = jnp.bitwise_and(t, 1)

        @pl.when(t + 1 < cnt)
        def _():
            in_copy(base + t + 1, 1 - sl).start()

        in_copy(base + t, sl).wait()

        @pl.when(t >= 2)
        def _():
            out_copy(base + t - 2, sl).wait()

        def tv(v, inner):
            vv = jnp.broadcast_to(v, (L,)).astype(jnp.int32)
            for l in range(NVR):
                g = plsc.load_gather(slab_v.at[sl], [fidx[l], vv])
                tslab_v[sl, pl.ds(v * D + L * l, L)] = g
            return inner

        lax.fori_loop(0, TBLK, tv, 0, unroll=2)
        out_copy(base + t, sl).start()
        return carry

    lax.fori_loop(0, cnt, blk, 0)

    @pl.when(cnt >= 2)
    def _():
        out_copy(base + cnt - 2, jnp.bitwise_and(cnt - 2, 1)).wait()

    @pl.when(cnt >= 1)
    def _():
        out_copy(base + cnt - 1, jnp.bitwise_and(cnt - 1, 1)).wait()

    @pl.when(wid == 0)
    def _():
        pltpu.sync_copy(tail_hbm, tail_v)
        pltpu.sync_copy(tail_v, out_hbm.at[pl.ds(VTAIL * D, (V - VTAIL) * D)])


@functools.partial(
    pl.kernel,
    mesh=_mesh,
    out_type=jax.ShapeDtypeStruct((B, D), jnp.float32),
    scratch_types=[
        pltpu.VMEM((3, CB * S), jnp.int32),
        pltpu.VMEM((2, CB * S, D), jnp.float32),
        pltpu.VMEM((2, CB, D), jnp.float32),
        pltpu.SemaphoreType.DMA((3,)),
        pltpu.SemaphoreType.DMA((2,)),
        pltpu.SemaphoreType.DMA((2,)),
    ],
    compiler_params=pltpu.CompilerParams(use_tc_tiling_on_sc=False),
)
def _emb_pool(
    x_hbm, table_hbm, out_hbm, idx_v, rows_v, out_v, idx_sem, gat_sem, out_sem
):
    wid = lax.axis_index("s") * NC + lax.axis_index("c")
    lane = lax.broadcasted_iota(jnp.int32, (L,), 0)
    lo8 = jnp.minimum(jnp.maximum(8 - lane, 0), 1)
    hi8 = 1 - lo8

    def idx_copy(c, sl):
        return pltpu.make_async_copy(
            x_hbm.at[pl.ds((wid * BPW + c * CB) * S, CB * S)],
            idx_v.at[sl],
            idx_sem.at[sl],
        )

    def fire_gathers(bsl, gsl):
        for j in range(NG):
            pltpu.async_copy(
                table_hbm.at[idx_v.at[bsl, pl.ds(j * G, G)]],
                rows_v.at[gsl, pl.ds(j * G, G)],
                gat_sem.at[gsl],
            )

    def wait_gathers(bsl, gsl):
        for j in range(NG):
            pltpu.make_async_copy(
                table_hbm.at[idx_v.at[bsl, pl.ds(j * G, G)]],
                rows_v.at[gsl, pl.ds(j * G, G)],
                gat_sem.at[gsl],
            ).wait()

    idx_copy(0, 0).start()
    idx_copy(0, 0).wait()
    fire_gathers(0, 0)
    idx_copy(1, 1).start()

    def chunk_body(i, carry):
        row0 = wid * BPW + i * CB
        cur = jnp.bitwise_and(i, 1)
        nxt = jnp.bitwise_and(i + 1, 1)
        bsl = lax.rem(i, 3)
        bsl1 = lax.rem(i + 1, 3)
        bsl2 = lax.rem(i + 2, 3)

        @pl.when(i + 1 < NCH)
        def _():
            idx_copy(i + 1, bsl1).wait()
            fire_gathers(bsl1, nxt)

        @pl.when(i + 2 < NCH)
        def _():
            idx_copy(i + 2, bsl2).start()

        invs = []
        for r in range(CB):
            if r % 2 == 0:
                full0 = (S * r) // L
                shared_k = full0 + 12
                shared_mask = lo8
            else:
                shared_k = (S * r - 8) // L
                full0 = shared_k + 1
                shared_mask = hi8
            ones = (
                jnp.minimum(idx_v[bsl, pl.ds(L * shared_k, L)], 1) * shared_mask
            )
            for k in range(full0, full0 + 12):
                ones = ones + jnp.minimum(idx_v[bsl, pl.ds(L * k, L)], 1)
            cnt = ones[0]
            for j in range(1, L):
                cnt = cnt + ones[j]
            denom = jnp.maximum(
                jnp.broadcast_to(cnt, (L,)).astype(jnp.float32), 1.0
            )
            invs.append(1.0 / denom)

        wait_gathers(bsl, cur)

        @pl.when(i >= 2)
        def _():
            pltpu.make_async_copy(
                out_v.at[cur],
                out_hbm.at[pl.ds(row0 - 2 * CB, CB)],
                out_sem.at[cur],
            ).wait()

        for r in range(CB):
            def srow(s, accs):
                row = r * S + s
                return tuple(
                    a + rows_v[cur, row, pl.ds(L * l, L)]
                    for l, a in enumerate(accs)
                )

            accs = lax.fori_loop(
                0,
                S,
                srow,
                tuple(jnp.zeros((L,), jnp.float32) for _ in range(NVR)),
                unroll=8,
            )
            for l in range(NVR):
                out_v[cur, r, pl.ds(L * l, L)] = accs[l] * invs[r]

        pltpu.async_copy(
            out_v.at[cur], out_hbm.at[pl.ds(row0, CB)], out_sem.at[cur]
        )
        return carry

    lax.fori_loop(0, NCH, chunk_body, 0)

    for t in (NCH - 2, NCH - 1):
        pltpu.make_async_copy(
            out_v.at[t % 2],
            out_hbm.at[pl.ds(wid * BPW + t * CB, CB)],
            out_sem.at[t % 2],
        ).wait()


def kernel(x, table):
    tail = table[VTAIL:].reshape(-1)
    tlin = _table_lin(table.T, tail)
    return _emb_pool(x.reshape(-1), tlin.reshape(V, D))


# speedup vs baseline: 1.1736x; 1.1736x over previous
import functools

import jax
import jax.numpy as jnp
from jax import lax
from jax.experimental import pallas as pl
from jax.experimental.pallas import tpu as pltpu
from jax.experimental.pallas import tpu_sc as plsc

B = 16384
S = 200
D = 64
NC = 2
NS = 16
NW = NC * NS
BPW = B // NW
CB = 4
NCH = BPW // CB
G = 80
NG = (CB * S) // G
L = 16
NVR = D // L

_mesh = plsc.VectorSubcoreMesh(core_axis_name="c", subcore_axis_name="s")

V = 1000000
TBLK = 128
NBLK = V // TBLK
VTAIL = NBLK * TBLK
BLK_PER_W = NBLK // NW
BLK_REM = NBLK % NW


@functools.partial(
    pl.kernel,
    mesh=_mesh,
    out_type=jax.ShapeDtypeStruct((V * D,), jnp.float32),
    scratch_types=[
        pltpu.VMEM((2, D, TBLK), jnp.float32),
        pltpu.VMEM((2, TBLK * D), jnp.float32),
        pltpu.VMEM(((V - VTAIL) * D,), jnp.float32),
        pltpu.SemaphoreType.DMA((2,)),
        pltpu.SemaphoreType.DMA((2,)),
    ],
    compiler_params=pltpu.CompilerParams(
        use_tc_tiling_on_sc=True, needs_layout_passes=False
    ),
)
def _table_lin(tt_hbm, tail_hbm, out_hbm, slab_v, tslab_v, tail_v, in_sem, out_sem):
    wid = lax.axis_index("s") * NC + lax.axis_index("c")
    base = wid * BLK_PER_W + jnp.minimum(wid, BLK_REM)
    cnt = BLK_PER_W + jnp.where(wid < BLK_REM, 1, 0)
    viota64 = lax.broadcasted_iota(jnp.int32, (L,), 0) * D

    def in_copy(c, sl):
        return pltpu.make_async_copy(
            tt_hbm.at[:, pl.ds(c * TBLK, TBLK)], slab_v.at[sl], in_sem.at[sl]
        )

    def out_copy(c, sl):
        return pltpu.make_async_copy(
            tslab_v.at[sl],
            out_hbm.at[pl.ds(c * (TBLK * D), TBLK * D)],
            out_sem.at[sl],
        )

    in_copy(base, 0).start()

    def blk(t, carry):
        sl = jnp.bitwise_and(t, 1)

        @pl.when(t + 1 < cnt)
        def _():
            in_copy(base + t + 1, 1 - sl).start()

        in_copy(base + t, sl).wait()

        @pl.when(t >= 2)
        def _():
            out_copy(base + t - 2, sl).wait()

        slv = jnp.broadcast_to(sl, (L,)).astype(jnp.int32)

        def tf(f, inner):
            for m in range(TBLK // L):
                val = slab_v[sl, f, pl.ds(L * m, L)]
                idx = viota64 + (L * D * m + f)
                plsc.store_scatter(tslab_v, [slv, idx], val)
            return inner

        lax.fori_loop(0, D, tf, 0, unroll=2)
        out_copy(base + t, sl).start()
        return carry

    lax.fori_loop(0, cnt, blk, 0)

    @pl.when(cnt >= 2)
    def _():
        out_copy(base + cnt - 2, jnp.bitwise_and(cnt - 2, 1)).wait()

    @pl.when(cnt >= 1)
    def _():
        out_copy(base + cnt - 1, jnp.bitwise_and(cnt - 1, 1)).wait()

    @pl.when(wid == 0)
    def _():
        pltpu.sync_copy(tail_hbm, tail_v)
        pltpu.sync_copy(tail_v, out_hbm.at[pl.ds(VTAIL * D, (V - VTAIL) * D)])


@functools.partial(
    pl.kernel,
    mesh=_mesh,
    out_type=jax.ShapeDtypeStruct((B, D), jnp.float32),
    scratch_types=[
        pltpu.VMEM((3, CB * S), jnp.int32),
        pltpu.VMEM((2, CB * S, D), jnp.float32),
        pltpu.VMEM((2, CB, D), jnp.float32),
        pltpu.SemaphoreType.DMA((3,)),
        pltpu.SemaphoreType.DMA((2,)),
        pltpu.SemaphoreType.DMA((2,)),
    ],
    compiler_params=pltpu.CompilerParams(use_tc_tiling_on_sc=False),
)
def _emb_pool(
    x_hbm, table_hbm, out_hbm, idx_v, rows_v, out_v, idx_sem, gat_sem, out_sem
):
    wid = lax.axis_index("s") * NC + lax.axis_index("c")
    lane = lax.broadcasted_iota(jnp.int32, (L,), 0)
    lo8 = jnp.minimum(jnp.maximum(8 - lane, 0), 1)
    hi8 = 1 - lo8

    def idx_copy(c, sl):
        return pltpu.make_async_copy(
            x_hbm.at[pl.ds((wid * BPW + c * CB) * S, CB * S)],
            idx_v.at[sl],
            idx_sem.at[sl],
        )

    def fire_gathers(bsl, gsl):
        for j in range(NG):
            pltpu.async_copy(
                table_hbm.at[idx_v.at[bsl, pl.ds(j * G, G)]],
                rows_v.at[gsl, pl.ds(j * G, G)],
                gat_sem.at[gsl],
            )

    def wait_gathers(bsl, gsl):
        for j in range(NG):
            pltpu.make_async_copy(
                table_hbm.at[idx_v.at[bsl, pl.ds(j * G, G)]],
                rows_v.at[gsl, pl.ds(j * G, G)],
                gat_sem.at[gsl],
            ).wait()

    idx_copy(0, 0).start()
    idx_copy(0, 0).wait()
    fire_gathers(0, 0)
    idx_copy(1, 1).start()

    def chunk_body(i, carry):
        row0 = wid * BPW + i * CB
        cur = jnp.bitwise_and(i, 1)
        nxt = jnp.bitwise_and(i + 1, 1)
        bsl = lax.rem(i, 3)
        bsl1 = lax.rem(i + 1, 3)
        bsl2 = lax.rem(i + 2, 3)

        @pl.when(i + 1 < NCH)
        def _():
            idx_copy(i + 1, bsl1).wait()
            fire_gathers(bsl1, nxt)

        @pl.when(i + 2 < NCH)
        def _():
            idx_copy(i + 2, bsl2).start()

        invs = []
        for r in range(CB):
            if r % 2 == 0:
                full0 = (S * r) // L
                shared_k = full0 + 12
                shared_mask = lo8
            else:
                shared_k = (S * r - 8) // L
                full0 = shared_k + 1
                shared_mask = hi8
            ones = (
                jnp.minimum(idx_v[bsl, pl.ds(L * shared_k, L)], 1) * shared_mask
            )
            for k in range(full0, full0 + 12):
                ones = ones + jnp.minimum(idx_v[bsl, pl.ds(L * k, L)], 1)
            cnt = ones[0]
            for j in range(1, L):
                cnt = cnt + ones[j]
            denom = jnp.maximum(
                jnp.broadcast_to(cnt, (L,)).astype(jnp.float32), 1.0
            )
            invs.append(1.0 / denom)

        wait_gathers(bsl, cur)

        @pl.when(i >= 2)
        def _():
            pltpu.make_async_copy(
                out_v.at[cur],
                out_hbm.at[pl.ds(row0 - 2 * CB, CB)],
                out_sem.at[cur],
            ).wait()

        for r in range(CB):
            def srow(s, accs):
                row = r * S + s
                return tuple(
                    a + rows_v[cur, row, pl.ds(L * l, L)]
                    for l, a in enumerate(accs)
                )

            accs = lax.fori_loop(
                0,
                S,
                srow,
                tuple(jnp.zeros((L,), jnp.float32) for _ in range(NVR)),
                unroll=8,
            )
            for l in range(NVR):
                out_v[cur, r, pl.ds(L * l, L)] = accs[l] * invs[r]

        pltpu.async_copy(
            out_v.at[cur], out_hbm.at[pl.ds(row0, CB)], out_sem.at[cur]
        )
        return carry

    lax.fori_loop(0, NCH, chunk_body, 0)

    for t in (NCH - 2, NCH - 1):
        pltpu.make_async_copy(
            out_v.at[t % 2],
            out_hbm.at[pl.ds(wid * BPW + t * CB, CB)],
            out_sem.at[t % 2],
        ).wait()


def kernel(x, table):
    tail = table[VTAIL:].reshape(-1)
    tlin = _table_lin(table.T, tail)
    return _emb_pool(x.reshape(-1), tlin.reshape(V, D))


# speedup vs baseline: 1.2329x; 1.0505x over previous
import functools

import jax
import jax.numpy as jnp
from jax import lax
from jax.experimental import pallas as pl
from jax.experimental.pallas import tpu as pltpu
from jax.experimental.pallas import tpu_sc as plsc

B = 16384
S = 200
D = 64
NC = 2
NS = 16
NW = NC * NS
BPW = B // NW
CB = 4
NCH = BPW // CB
G = 80
NG = (CB * S) // G
L = 16
NVR = D // L

_mesh = plsc.VectorSubcoreMesh(core_axis_name="c", subcore_axis_name="s")

V = 1000000
TBLK = 128
NBLK = V // TBLK
VTAIL = NBLK * TBLK
BLK_PER_W = NBLK // NW
BLK_REM = NBLK % NW


@functools.partial(
    pl.kernel,
    mesh=_mesh,
    out_type=jax.ShapeDtypeStruct((V * D,), jnp.float32),
    scratch_types=[
        pltpu.VMEM((2, D, TBLK), jnp.float32),
        pltpu.VMEM((2 * TBLK * D,), jnp.float32),
        pltpu.VMEM(((V - VTAIL) * D,), jnp.float32),
        pltpu.SemaphoreType.DMA((2,)),
        pltpu.SemaphoreType.DMA((2,)),
    ],
    compiler_params=pltpu.CompilerParams(
        use_tc_tiling_on_sc=True, needs_layout_passes=False
    ),
)
def _table_lin(tt_hbm, tail_hbm, out_hbm, slab_v, tslab_v, tail_v, in_sem, out_sem):
    wid = lax.axis_index("s") * NC + lax.axis_index("c")
    base = wid * BLK_PER_W + jnp.minimum(wid, BLK_REM)
    NPAIR = BLK_PER_W // 2
    viota64 = lax.broadcasted_iota(jnp.int32, (L,), 0) * D
    bidx = [
        [viota64 + (L * D * m + s * TBLK * D) for m in range(TBLK // L)]
        for s in range(2)
    ]

    def in_copy(c, s):
        return pltpu.make_async_copy(
            tt_hbm.at[:, pl.ds(c * TBLK, TBLK)], slab_v.at[s], in_sem.at[s]
        )

    def out_copy(c, s):
        return pltpu.make_async_copy(
            tslab_v.at[pl.ds(s * TBLK * D, TBLK * D)],
            out_hbm.at[pl.ds(c * (TBLK * D), TBLK * D)],
            out_sem.at[s],
        )

    def transpose_slot(s):
        def tf(f, inner):
            for m in range(TBLK // L):
                val = slab_v[s, f, pl.ds(L * m, L)]
                plsc.store_scatter(tslab_v, [bidx[s][m] + f], val)
            return inner

        lax.fori_loop(0, D, tf, 0, unroll=2)

    in_copy(base, 0).start()

    def pair_body(t, carry):
        c0 = base + 2 * t

        in_copy(c0 + 1, 1).start()
        in_copy(c0, 0).wait()

        @pl.when(t >= 1)
        def _():
            out_copy(c0 - 2, 0).wait()

        transpose_slot(0)
        out_copy(c0, 0).start()

        @pl.when(t + 1 < NPAIR)
        def _():
            in_copy(c0 + 2, 0).start()

        in_copy(c0 + 1, 1).wait()

        @pl.when(t >= 1)
        def _():
            out_copy(c0 - 1, 1).wait()

        transpose_slot(1)
        out_copy(c0 + 1, 1).start()
        return carry

    lax.fori_loop(0, NPAIR, pair_body, 0)

    out_copy(base + 2 * NPAIR - 2, 0).wait()
    out_copy(base + 2 * NPAIR - 1, 1).wait()

    @pl.when(wid < BLK_REM)
    def _():
        c = base + 2 * NPAIR
        in_copy(c, 0).start()
        in_copy(c, 0).wait()
        transpose_slot(0)
        out_copy(c, 0).start()
        out_copy(c, 0).wait()

    @pl.when(wid == 0)
    def _():
        pltpu.sync_copy(tail_hbm, tail_v)
        pltpu.sync_copy(tail_v, out_hbm.at[pl.ds(VTAIL * D, (V - VTAIL) * D)])


@functools.partial(
    pl.kernel,
    mesh=_mesh,
    out_type=jax.ShapeDtypeStruct((B, D), jnp.float32),
    scratch_types=[
        pltpu.VMEM((3, CB * S), jnp.int32),
        pltpu.VMEM((2, CB * S, D), jnp.float32),
        pltpu.VMEM((2, CB, D), jnp.float32),
        pltpu.SemaphoreType.DMA((3,)),
        pltpu.SemaphoreType.DMA((2,)),
        pltpu.SemaphoreType.DMA((2,)),
    ],
    compiler_params=pltpu.CompilerParams(use_tc_tiling_on_sc=False),
)
def _emb_pool(
    x_hbm, table_hbm, out_hbm, idx_v, rows_v, out_v, idx_sem, gat_sem, out_sem
):
    wid = lax.axis_index("s") * NC + lax.axis_index("c")
    lane = lax.broadcasted_iota(jnp.int32, (L,), 0)
    lo8 = jnp.minimum(jnp.maximum(8 - lane, 0), 1)
    hi8 = 1 - lo8

    def idx_copy(c, sl):
        return pltpu.make_async_copy(
            x_hbm.at[pl.ds((wid * BPW + c * CB) * S, CB * S)],
            idx_v.at[sl],
            idx_sem.at[sl],
        )

    def fire_gathers(bsl, gsl):
        for j in range(NG):
            pltpu.async_copy(
                table_hbm.at[idx_v.at[bsl, pl.ds(j * G, G)]],
                rows_v.at[gsl, pl.ds(j * G, G)],
                gat_sem.at[gsl],
            )

    def wait_gathers(bsl, gsl):
        for j in range(NG):
            pltpu.make_async_copy(
                table_hbm.at[idx_v.at[bsl, pl.ds(j * G, G)]],
                rows_v.at[gsl, pl.ds(j * G, G)],
                gat_sem.at[gsl],
            ).wait()

    idx_copy(0, 0).start()
    idx_copy(0, 0).wait()
    fire_gathers(0, 0)
    idx_copy(1, 1).start()

    def chunk_body(i, carry):
        row0 = wid * BPW + i * CB
        cur = jnp.bitwise_and(i, 1)
        nxt = jnp.bitwise_and(i + 1, 1)
        bsl = lax.rem(i, 3)
        bsl1 = lax.rem(i + 1, 3)
        bsl2 = lax.rem(i + 2, 3)

        @pl.when(i + 1 < NCH)
        def _():
            idx_copy(i + 1, bsl1).wait()
            fire_gathers(bsl1, nxt)

        @pl.when(i + 2 < NCH)
        def _():
            idx_copy(i + 2, bsl2).start()

        invs = []
        for r in range(CB):
            if r % 2 == 0:
                full0 = (S * r) // L
                shared_k = full0 + 12
                shared_mask = lo8
            else:
                shared_k = (S * r - 8) // L
                full0 = shared_k + 1
                shared_mask = hi8
            ones = (
                jnp.minimum(idx_v[bsl, pl.ds(L * shared_k, L)], 1) * shared_mask
            )
            for k in range(full0, full0 + 12):
                ones = ones + jnp.minimum(idx_v[bsl, pl.ds(L * k, L)], 1)
            cnt = ones[0]
            for j in range(1, L):
                cnt = cnt + ones[j]
            denom = jnp.maximum(
                jnp.broadcast_to(cnt, (L,)).astype(jnp.float32), 1.0
            )
            invs.append(1.0 / denom)

        wait_gathers(bsl, cur)

        @pl.when(i >= 2)
        def _():
            pltpu.make_async_copy(
                out_v.at[cur],
                out_hbm.at[pl.ds(row0 - 2 * CB, CB)],
                out_sem.at[cur],
            ).wait()

        for r in range(CB):
            def srow(s, accs):
                row = r * S + s
                return tuple(
                    a + rows_v[cur, row, pl.ds(L * l, L)]
                    for l, a in enumerate(accs)
                )

            accs = lax.fori_loop(
                0,
                S,
                srow,
                tuple(jnp.zeros((L,), jnp.float32) for _ in range(NVR)),
                unroll=8,
            )
            for l in range(NVR):
                out_v[cur, r, pl.ds(L * l, L)] = accs[l] * invs[r]

        pltpu.async_copy(
            out_v.at[cur], out_hbm.at[pl.ds(row0, CB)], out_sem.at[cur]
        )
        return carry

    lax.fori_loop(0, NCH, chunk_body, 0)

    for t in (NCH - 2, NCH - 1):
        pltpu.make_async_copy(
            out_v.at[t % 2],
            out_hbm.at[pl.ds(wid * BPW + t * CB, CB)],
            out_sem.at[t % 2],
        ).wait()


def kernel(x, table):
    tail = table[VTAIL:].reshape(-1)
    tlin = _table_lin(table.T, tail)
    return _emb_pool(x.reshape(-1), tlin.reshape(V, D))


# speedup vs baseline: 2.2372x; 1.8145x over previous
import functools

import jax
import jax.numpy as jnp
from jax import lax
from jax.experimental import pallas as pl
from jax.experimental.pallas import tpu as pltpu
from jax.experimental.pallas import tpu_sc as plsc

B = 16384
S = 200
D = 64
NC = 2
NS = 16
NW = NC * NS
BPW = B // NW
CB = 4
NCH = BPW // CB
G = 80
NG = (CB * S) // G
L = 16
NVR = D // L

_mesh = plsc.VectorSubcoreMesh(core_axis_name="c", subcore_axis_name="s")

V = 1000000
TBLK = 128
NBLK = V // TBLK
VTAIL = NBLK * TBLK
BLK_PER_W = NBLK // NW
BLK_REM = NBLK % NW


@functools.partial(
    pl.kernel,
    mesh=_mesh,
    out_type=jax.ShapeDtypeStruct((V * D,), jnp.float32),
    scratch_types=[
        pltpu.VMEM((2 * D, TBLK), jnp.float32),
        pltpu.VMEM((2 * TBLK * D,), jnp.float32),
        pltpu.VMEM(((V - VTAIL) * D,), jnp.float32),
        pltpu.SemaphoreType.DMA((2,)),
        pltpu.SemaphoreType.DMA((2,)),
    ],
    compiler_params=pltpu.CompilerParams(
        use_tc_tiling_on_sc=True, needs_layout_passes=False
    ),
)
def _table_lin(tt_hbm, tail_hbm, out_hbm, slab_v, tslab_v, tail_v, in_sem, out_sem):
    wid = lax.axis_index("s") * NC + lax.axis_index("c")
    base = wid * BLK_PER_W + jnp.minimum(wid, BLK_REM)
    NPAIR = BLK_PER_W // 2
    lane16 = lax.broadcasted_iota(jnp.int32, (L,), 0)
    cm = [lane16 + L * m for m in range(TBLK // L)]
    c2 = [
        [(lane16 + L * m) * D + s * TBLK * D for m in range(TBLK // L)]
        for s in range(2)
    ]

    def in_copy(c, s):
        return pltpu.make_async_copy(
            tt_hbm.at[:, pl.ds(c * TBLK, TBLK)],
            slab_v.at[pl.ds(s * D, D), :],
            in_sem.at[s],
        )

    def out_copy(c, s):
        return pltpu.make_async_copy(
            tslab_v.at[pl.ds(s * TBLK * D, TBLK * D)],
            out_hbm.at[pl.ds(c * (TBLK * D), TBLK * D)],
            out_sem.at[s],
        )

    def transpose_slot(s):
        def tf(f0, inner):
            fq = jnp.bitwise_and(f0 + lane16, D - 1)
            frow = fq + s * D
            for m in range(TBLK // L):
                val = plsc.load_gather(slab_v, [frow, cm[m]])
                plsc.store_scatter(tslab_v, [c2[s][m] + fq], val)
            return inner

        lax.fori_loop(0, D, tf, 0, unroll=2)

    in_copy(base, 0).start()

    def pair_body(t, carry):
        c0 = base + 2 * t

        in_copy(c0 + 1, 1).start()
        in_copy(c0, 0).wait()

        @pl.when(t >= 1)
        def _():
            out_copy(c0 - 2, 0).wait()

        transpose_slot(0)
        out_copy(c0, 0).start()

        @pl.when(t + 1 < NPAIR)
        def _():
            in_copy(c0 + 2, 0).start()

        in_copy(c0 + 1, 1).wait()

        @pl.when(t >= 1)
        def _():
            out_copy(c0 - 1, 1).wait()

        transpose_slot(1)
        out_copy(c0 + 1, 1).start()
        return carry

    lax.fori_loop(0, NPAIR, pair_body, 0)

    out_copy(base + 2 * NPAIR - 2, 0).wait()
    out_copy(base + 2 * NPAIR - 1, 1).wait()

    @pl.when(wid < BLK_REM)
    def _():
        c = base + 2 * NPAIR
        in_copy(c, 0).start()
        in_copy(c, 0).wait()
        transpose_slot(0)
        out_copy(c, 0).start()
        out_copy(c, 0).wait()

    @pl.when(wid == 0)
    def _():
        pltpu.sync_copy(tail_hbm, tail_v)
        pltpu.sync_copy(tail_v, out_hbm.at[pl.ds(VTAIL * D, (V - VTAIL) * D)])


@functools.partial(
    pl.kernel,
    mesh=_mesh,
    out_type=jax.ShapeDtypeStruct((B, D), jnp.float32),
    scratch_types=[
        pltpu.VMEM((3, CB * S), jnp.int32),
        pltpu.VMEM((2, CB * S, D), jnp.float32),
        pltpu.VMEM((2, CB, D), jnp.float32),
        pltpu.SemaphoreType.DMA((3,)),
        pltpu.SemaphoreType.DMA((2,)),
        pltpu.SemaphoreType.DMA((2,)),
    ],
    compiler_params=pltpu.CompilerParams(use_tc_tiling_on_sc=False),
)
def _emb_pool(
    x_hbm, table_hbm, out_hbm, idx_v, rows_v, out_v, idx_sem, gat_sem, out_sem
):
    wid = lax.axis_index("s") * NC + lax.axis_index("c")
    lane = lax.broadcasted_iota(jnp.int32, (L,), 0)
    lo8 = jnp.minimum(jnp.maximum(8 - lane, 0), 1)
    hi8 = 1 - lo8

    def idx_copy(c, sl):
        return pltpu.make_async_copy(
            x_hbm.at[pl.ds((wid * BPW + c * CB) * S, CB * S)],
            idx_v.at[sl],
            idx_sem.at[sl],
        )

    def fire_gathers(bsl, gsl):
        for j in range(NG):
            pltpu.async_copy(
                table_hbm.at[idx_v.at[bsl, pl.ds(j * G, G)]],
                rows_v.at[gsl, pl.ds(j * G, G)],
                gat_sem.at[gsl],
            )

    def wait_gathers(bsl, gsl):
        for j in range(NG):
            pltpu.make_async_copy(
                table_hbm.at[idx_v.at[bsl, pl.ds(j * G, G)]],
                rows_v.at[gsl, pl.ds(j * G, G)],
                gat_sem.at[gsl],
            ).wait()

    idx_copy(0, 0).start()
    idx_copy(0, 0).wait()
    fire_gathers(0, 0)
    idx_copy(1, 1).start()

    def chunk_body(i, carry):
        row0 = wid * BPW + i * CB
        cur = jnp.bitwise_and(i, 1)
        nxt = jnp.bitwise_and(i + 1, 1)
        bsl = lax.rem(i, 3)
        bsl1 = lax.rem(i + 1, 3)
        bsl2 = lax.rem(i + 2, 3)

        @pl.when(i + 1 < NCH)
        def _():
            idx_copy(i + 1, bsl1).wait()
            fire_gathers(bsl1, nxt)

        @pl.when(i + 2 < NCH)
        def _():
            idx_copy(i + 2, bsl2).start()

        invs = []
        for r in range(CB):
            if r % 2 == 0:
                full0 = (S * r) // L
                shared_k = full0 + 12
                shared_mask = lo8
            else:
                shared_k = (S * r - 8) // L
                full0 = shared_k + 1
                shared_mask = hi8
            ones = (
                jnp.minimum(idx_v[bsl, pl.ds(L * shared_k, L)], 1) * shared_mask
            )
            for k in range(full0, full0 + 12):
                ones = ones + jnp.minimum(idx_v[bsl, pl.ds(L * k, L)], 1)
            cnt = ones[0]
            for j in range(1, L):
                cnt = cnt + ones[j]
            denom = jnp.maximum(
                jnp.broadcast_to(cnt, (L,)).astype(jnp.float32), 1.0
            )
            invs.append(1.0 / denom)

        wait_gathers(bsl, cur)

        @pl.when(i >= 2)
        def _():
            pltpu.make_async_copy(
                out_v.at[cur],
                out_hbm.at[pl.ds(row0 - 2 * CB, CB)],
                out_sem.at[cur],
            ).wait()

        for r in range(CB):
            def srow(s, accs):
                row = r * S + s
                return tuple(
                    a + rows_v[cur, row, pl.ds(L * l, L)]
                    for l, a in enumerate(accs)
                )

            accs = lax.fori_loop(
                0,
                S,
                srow,
                tuple(jnp.zeros((L,), jnp.float32) for _ in range(NVR)),
                unroll=8,
            )
            for l in range(NVR):
                out_v[cur, r, pl.ds(L * l, L)] = accs[l] * invs[r]

        pltpu.async_copy(
            out_v.at[cur], out_hbm.at[pl.ds(row0, CB)], out_sem.at[cur]
        )
        return carry

    lax.fori_loop(0, NCH, chunk_body, 0)

    for t in (NCH - 2, NCH - 1):
        pltpu.make_async_copy(
            out_v.at[t % 2],
            out_hbm.at[pl.ds(wid * BPW + t * CB, CB)],
            out_sem.at[t % 2],
        ).wait()


def kernel(x, table):
    tail = table[VTAIL:].reshape(-1)
    tlin = _table_lin(table.T, tail)
    return _emb_pool(x.reshape(-1), tlin.reshape(V, D))


# speedup vs baseline: 3.3525x; 1.4985x over previous
import functools

import jax
import jax.numpy as jnp
from jax import lax
from jax.experimental import pallas as pl
from jax.experimental.pallas import tpu as pltpu
from jax.experimental.pallas import tpu_sc as plsc

B = 16384
S = 200
D = 64
NC = 2
NS = 16
NW = NC * NS
BPW = B // NW
CB = 4
NCH = BPW // CB
G = 80
NG = (CB * S) // G
L = 16
NVR = D // L

_mesh = plsc.VectorSubcoreMesh(core_axis_name="c", subcore_axis_name="s")

V = 1000000
TBLK = 128
NBLK = V // TBLK
VTAIL = NBLK * TBLK
BLK_PER_W = NBLK // NW
BLK_REM = NBLK % NW


@functools.partial(
    pl.kernel,
    mesh=_mesh,
    out_type=jax.ShapeDtypeStruct((V * D,), jnp.float32),
    scratch_types=[
        pltpu.VMEM((2 * D, TBLK), jnp.float32),
        pltpu.VMEM((2 * TBLK * D,), jnp.float32),
        pltpu.VMEM(((V - VTAIL) * D,), jnp.float32),
        pltpu.SemaphoreType.DMA((2,)),
        pltpu.SemaphoreType.DMA((2,)),
    ],
    compiler_params=pltpu.CompilerParams(
        use_tc_tiling_on_sc=True, needs_layout_passes=False
    ),
)
def _table_lin(tt_hbm, tail_hbm, out_hbm, slab_v, tslab_v, tail_v, in_sem, out_sem):
    wid = lax.axis_index("s") * NC + lax.axis_index("c")
    base = wid * BLK_PER_W + jnp.minimum(wid, BLK_REM)
    NPAIR = BLK_PER_W // 2
    lane16 = lax.broadcasted_iota(jnp.int32, (L,), 0)
    cm = [lane16 + L * m for m in range(TBLK // L)]
    c2 = [
        [(lane16 + L * m) * D + s * TBLK * D for m in range(TBLK // L)]
        for s in range(2)
    ]

    def in_copy(c, s):
        return pltpu.make_async_copy(
            tt_hbm.at[:, pl.ds(c * TBLK, TBLK)],
            slab_v.at[pl.ds(s * D, D), :],
            in_sem.at[s],
        )

    def out_copy(c, s):
        return pltpu.make_async_copy(
            tslab_v.at[pl.ds(s * TBLK * D, TBLK * D)],
            out_hbm.at[pl.ds(c * (TBLK * D), TBLK * D)],
            out_sem.at[s],
        )

    def transpose_slot(s):
        @plsc.parallel_loop(0, D, unroll=4)
        def tf(f0):
            fq = jnp.bitwise_and(f0 + lane16, D - 1)
            frow = fq + s * D
            for m in range(TBLK // L):
                val = plsc.load_gather(slab_v, [frow, cm[m]])
                plsc.store_scatter(tslab_v, [c2[s][m] + fq], val)

    in_copy(base, 0).start()

    def pair_body(t, carry):
        c0 = base + 2 * t

        in_copy(c0 + 1, 1).start()
        in_copy(c0, 0).wait()

        @pl.when(t >= 1)
        def _():
            out_copy(c0 - 2, 0).wait()

        transpose_slot(0)
        out_copy(c0, 0).start()

        @pl.when(t + 1 < NPAIR)
        def _():
            in_copy(c0 + 2, 0).start()

        in_copy(c0 + 1, 1).wait()

        @pl.when(t >= 1)
        def _():
            out_copy(c0 - 1, 1).wait()

        transpose_slot(1)
        out_copy(c0 + 1, 1).start()
        return carry

    lax.fori_loop(0, NPAIR, pair_body, 0)

    out_copy(base + 2 * NPAIR - 2, 0).wait()
    out_copy(base + 2 * NPAIR - 1, 1).wait()

    @pl.when(wid < BLK_REM)
    def _():
        c = base + 2 * NPAIR
        in_copy(c, 0).start()
        in_copy(c, 0).wait()
        transpose_slot(0)
        out_copy(c, 0).start()
        out_copy(c, 0).wait()

    @pl.when(wid == 0)
    def _():
        pltpu.sync_copy(tail_hbm, tail_v)
        pltpu.sync_copy(tail_v, out_hbm.at[pl.ds(VTAIL * D, (V - VTAIL) * D)])


@functools.partial(
    pl.kernel,
    mesh=_mesh,
    out_type=jax.ShapeDtypeStruct((B, D), jnp.float32),
    scratch_types=[
        pltpu.VMEM((3, CB * S), jnp.int32),
        pltpu.VMEM((2, CB * S, D), jnp.float32),
        pltpu.VMEM((2, CB, D), jnp.float32),
        pltpu.SemaphoreType.DMA((3,)),
        pltpu.SemaphoreType.DMA((2,)),
        pltpu.SemaphoreType.DMA((2,)),
    ],
    compiler_params=pltpu.CompilerParams(use_tc_tiling_on_sc=False),
)
def _emb_pool(
    x_hbm, table_hbm, out_hbm, idx_v, rows_v, out_v, idx_sem, gat_sem, out_sem
):
    wid = lax.axis_index("s") * NC + lax.axis_index("c")
    lane = lax.broadcasted_iota(jnp.int32, (L,), 0)
    lo8 = jnp.minimum(jnp.maximum(8 - lane, 0), 1)
    hi8 = 1 - lo8

    def idx_copy(c, sl):
        return pltpu.make_async_copy(
            x_hbm.at[pl.ds((wid * BPW + c * CB) * S, CB * S)],
            idx_v.at[sl],
            idx_sem.at[sl],
        )

    def fire_gathers(bsl, gsl):
        for j in range(NG):
            pltpu.async_copy(
                table_hbm.at[idx_v.at[bsl, pl.ds(j * G, G)]],
                rows_v.at[gsl, pl.ds(j * G, G)],
                gat_sem.at[gsl],
            )

    def wait_gathers(bsl, gsl):
        for j in range(NG):
            pltpu.make_async_copy(
                table_hbm.at[idx_v.at[bsl, pl.ds(j * G, G)]],
                rows_v.at[gsl, pl.ds(j * G, G)],
                gat_sem.at[gsl],
            ).wait()

    idx_copy(0, 0).start()
    idx_copy(0, 0).wait()
    fire_gathers(0, 0)
    idx_copy(1, 1).start()

    def chunk_body(i, carry):
        row0 = wid * BPW + i * CB
        cur = jnp.bitwise_and(i, 1)
        nxt = jnp.bitwise_and(i + 1, 1)
        bsl = lax.rem(i, 3)
        bsl1 = lax.rem(i + 1, 3)
        bsl2 = lax.rem(i + 2, 3)

        @pl.when(i + 1 < NCH)
        def _():
            idx_copy(i + 1, bsl1).wait()
            fire_gathers(bsl1, nxt)

        @pl.when(i + 2 < NCH)
        def _():
            idx_copy(i + 2, bsl2).start()

        invs = []
        for r in range(CB):
            if r % 2 == 0:
                full0 = (S * r) // L
                shared_k = full0 + 12
                shared_mask = lo8
            else:
                shared_k = (S * r - 8) // L
                full0 = shared_k + 1
                shared_mask = hi8
            ones = (
                jnp.minimum(idx_v[bsl, pl.ds(L * shared_k, L)], 1) * shared_mask
            )
            for k in range(full0, full0 + 12):
                ones = ones + jnp.minimum(idx_v[bsl, pl.ds(L * k, L)], 1)
            cnt = ones[0]
            for j in range(1, L):
                cnt = cnt + ones[j]
            denom = jnp.maximum(
                jnp.broadcast_to(cnt, (L,)).astype(jnp.float32), 1.0
            )
            invs.append(1.0 / denom)

        wait_gathers(bsl, cur)

        @pl.when(i >= 2)
        def _():
            pltpu.make_async_copy(
                out_v.at[cur],
                out_hbm.at[pl.ds(row0 - 2 * CB, CB)],
                out_sem.at[cur],
            ).wait()

        for r in range(CB):
            def srow(s, accs):
                row = r * S + s
                return tuple(
                    a + rows_v[cur, row, pl.ds(L * l, L)]
                    for l, a in enumerate(accs)
                )

            accs = lax.fori_loop(
                0,
                S,
                srow,
                tuple(jnp.zeros((L,), jnp.float32) for _ in range(NVR)),
                unroll=8,
            )
            for l in range(NVR):
                out_v[cur, r, pl.ds(L * l, L)] = accs[l] * invs[r]

        pltpu.async_copy(
            out_v.at[cur], out_hbm.at[pl.ds(row0, CB)], out_sem.at[cur]
        )
        return carry

    lax.fori_loop(0, NCH, chunk_body, 0)

    for t in (NCH - 2, NCH - 1):
        pltpu.make_async_copy(
            out_v.at[t % 2],
            out_hbm.at[pl.ds(wid * BPW + t * CB, CB)],
            out_sem.at[t % 2],
        ).wait()


def kernel(x, table):
    tail = table[VTAIL:].reshape(-1)
    tlin = _table_lin(table.T, tail)
    return _emb_pool(x.reshape(-1), tlin.reshape(V, D))
